# baseline (speedup 1.0000x reference)
import functools

import jax
import jax.numpy as jnp
from jax import lax
from jax.experimental import pallas as pl
from jax.experimental.pallas import tpu as pltpu
from jax.experimental.pallas import tpu_sc as plsc


def _add_row_kernel(x_ref, w_ref, o_ref):
    o_ref[...] = x_ref[...] + w_ref[...]


def _kernel_tc(x, W):
    B, L, D = x.shape
    patch = L // 3
    xr = x.reshape(B * 3, patch, D)
    Wr = W.reshape(3, 1, D)

    BL = 2048
    grid = (B * 3, patch // BL)

    out = pl.pallas_call(
        _add_row_kernel,
        grid=grid,
        in_specs=[
            pl.BlockSpec((1, BL, D), lambda i, j: (i, j, 0)),
            pl.BlockSpec((1, 1, D), lambda i, j: (i % 3, 0, 0)),
        ],
        out_specs=pl.BlockSpec((1, BL, D), lambda i, j: (i, j, 0)),
        out_shape=jax.ShapeDtypeStruct((B * 3, patch, D), x.dtype),
    )(xr, Wr)
    return out.reshape(B, L, D)


_NW = 32
_CH = 32


def _kernel_sc(x, W):
    B, L, D = x.shape
    patch = L // 3
    R = B * L
    band = patch // _NW
    nchunk = band // _CH
    xf = x.reshape(R * D)
    wf = W.reshape(3 * D)

    mesh = plsc.VectorSubcoreMesh(core_axis_name="c", subcore_axis_name="s")

    @functools.partial(
        pl.kernel,
        mesh=mesh,
        out_type=jax.ShapeDtypeStruct((R * D,), jnp.float32),
        scratch_types=[
            pltpu.VMEM((3 * D,), jnp.float32),
            pltpu.VMEM((_CH * D,), jnp.float32),
        ],
    )
    def sc_add(x_hbm, w_hbm, o_hbm, w_v, acc_v):
        wid = lax.axis_index("s") * 2 + lax.axis_index("c")
        pltpu.sync_copy(w_hbm, w_v)
        for seg in range(3):
            @pl.loop(0, B * nchunk)
            def _chunks(k):
                b = k // nchunk
                c = lax.rem(k, nchunk)
                row0 = (b * 3 + seg) * patch + wid * band + c * _CH
                base = row0 * D
                pltpu.sync_copy(x_hbm.at[pl.ds(base, _CH * D)], acc_v)

                @plsc.parallel_loop(0, _CH * (D // 16), unroll=8)
                def _add(i):
                    j = lax.rem(i, D // 16)
                    sl = pl.ds(i * 16, 16)
                    acc_v[sl] = acc_v[sl] + w_v[pl.ds(seg * D + j * 16, 16)]

                pltpu.sync_copy(acc_v, o_hbm.at[pl.ds(base, _CH * D)])

    out = sc_add(xf, wf)
    return out.reshape(B, L, D)


def kernel(x, W):
    return _kernel_sc(x, W)


# speedup vs baseline: 1.4537x; 1.4537x over previous
import functools

import jax
import jax.numpy as jnp
from jax import lax
from jax.experimental import pallas as pl
from jax.experimental.pallas import tpu as pltpu
from jax.experimental.pallas import tpu_sc as plsc


def _add_row_kernel(x_ref, w_ref, o_ref):
    o_ref[...] = x_ref[...] + w_ref[...]


def _kernel_tc(x, W):
    B, L, D = x.shape
    patch = L // 3
    xr = x.reshape(B * 3, patch, D)
    Wr = W.reshape(3, 1, D)

    BL = 2048
    grid = (B * 3, patch // BL)

    out = pl.pallas_call(
        _add_row_kernel,
        grid=grid,
        in_specs=[
            pl.BlockSpec((1, BL, D), lambda i, j: (i, j, 0)),
            pl.BlockSpec((1, 1, D), lambda i, j: (i % 3, 0, 0)),
        ],
        out_specs=pl.BlockSpec((1, BL, D), lambda i, j: (i, j, 0)),
        out_shape=jax.ShapeDtypeStruct((B * 3, patch, D), x.dtype),
    )(xr, Wr)
    return out.reshape(B, L, D)


_NW = 32
_CH = 32


def _kernel_sc(x, W):
    B, L, D = x.shape
    patch = L // 3
    R = B * L
    band = patch // _NW
    nchunk = band // _CH
    xf = x.reshape(R * D)
    wf = W.reshape(3 * D)

    mesh = plsc.VectorSubcoreMesh(core_axis_name="c", subcore_axis_name="s")

    @functools.partial(
        pl.kernel,
        mesh=mesh,
        out_type=jax.ShapeDtypeStruct((R * D,), jnp.float32),
        scratch_types=[
            pltpu.VMEM((3 * D,), jnp.float32),
            pltpu.VMEM((_CH * D,), jnp.float32),
        ],
    )
    def sc_add(x_hbm, w_hbm, o_hbm, w_v, acc_v):
        wid = lax.axis_index("s") * 2 + lax.axis_index("c")
        pltpu.sync_copy(w_hbm, w_v)
        for seg in range(3):
            @pl.loop(0, B * nchunk)
            def _chunks(k):
                b = k // nchunk
                c = lax.rem(k, nchunk)
                row0 = (b * 3 + seg) * patch + wid * band + c * _CH
                base = row0 * D
                pltpu.sync_copy(x_hbm.at[pl.ds(base, _CH * D)], acc_v)

                @plsc.parallel_loop(0, _CH * (D // 16), unroll=8)
                def _add(i):
                    j = lax.rem(i, D // 16)
                    sl = pl.ds(i * 16, 16)
                    acc_v[sl] = acc_v[sl] + w_v[pl.ds(seg * D + j * 16, 16)]

                pltpu.sync_copy(acc_v, o_hbm.at[pl.ds(base, _CH * D)])

    out = sc_add(xf, wf)
    return out.reshape(B, L, D)


_B_SC = 1


def kernel(x, W):
    B = x.shape[0]
    out_tc = _kernel_tc(x[: B - _B_SC], W)
    out_sc = _kernel_sc(x[B - _B_SC :], W)
    return jnp.concatenate([out_tc, out_sc], axis=0)
